# baseline (speedup 1.0000x reference)
import functools

import jax
import jax.numpy as jnp
from jax import lax
from jax.experimental import pallas as pl
from jax.experimental.pallas import tpu as pltpu
from jax.experimental.pallas import tpu_sc as plsc

N_VOTERS = 64
BATCH = 1048576
LANES = 16
NUM_WORKERS = 32

SC_COLS = 1048576
TC_COLS = BATCH - SC_COLS
CHUNK = 512
NSL = CHUNK // LANES
BLK_COLS = 16 * CHUNK
BPC = max(SC_COLS // 2, BLK_COLS)
NCH = BPC // BLK_COLS
TC_BLK = 65536

_MESH = plsc.VectorSubcoreMesh(core_axis_name="c", subcore_axis_name="s")


@functools.partial(
    pl.kernel,
    out_type=jax.ShapeDtypeStruct((max(SC_COLS, 1),), jnp.int32),
    mesh=_MESH,
    scratch_types=[
        pltpu.VMEM((N_VOTERS, LANES), jnp.float32),
        pltpu.VMEM_SHARED((N_VOTERS, BLK_COLS), jnp.int32),
        pltpu.VMEM_SHARED((N_VOTERS, BLK_COLS), jnp.int32),
        pltpu.VMEM((N_VOTERS, CHUNK), jnp.int32),
        pltpu.VMEM((CHUNK,), jnp.int32),
        pltpu.VMEM((CHUNK,), jnp.int32),
        pltpu.SemaphoreType.DMA,
        pltpu.SemaphoreType.DMA,
        pltpu.SemaphoreType.DMA,
    ],
)
def _vote_sc(votes_hbm, wb_hbm, out_hbm, wb_v, sh0, sh1, buf, out_v0,
             out_v1, hsem, osem0, osem1):
    cid = lax.axis_index("c")
    sid = lax.axis_index("s")
    cbase = cid * BPC
    tbase = cbase + sid * CHUNK

    pltpu.sync_copy(wb_hbm, wb_v)

    def _wsum(v, s):
        return s + wb_v[v]
    sumw = lax.fori_loop(0, N_VOTERS, _wsum,
                         jnp.zeros((LANES,), jnp.float32))

    def _start(g, sh):
        pltpu.async_copy(
            votes_hbm.at[:, pl.ds(cbase + g * BLK_COLS, BLK_COLS)], sh, hsem)

    def _hwait(sh):
        pltpu.make_async_copy(
            votes_hbm.at[:, pl.ds(cbase, BLK_COLS)], sh, hsem).wait()

    def _owait(out_v, osem):
        pltpu.make_async_copy(
            out_v, out_hbm.at[pl.ds(tbase, CHUNK)], osem).wait()

    def _compute(g, sh, out_v, osem):
        pltpu.sync_copy(sh.at[:, pl.ds(sid * CHUNK, CHUNK)], buf)

        def _vstep(v, accs):
            w = wb_v[v]
            return tuple(
                accs[sl] + buf[v, pl.ds(sl * LANES, LANES)]
                .astype(jnp.float32) * w
                for sl in range(NSL))
        zero = jnp.zeros((LANES,), jnp.float32)
        accs = lax.fori_loop(0, N_VOTERS, _vstep, (zero,) * NSL)

        @pl.when(g >= 2)
        def _():
            _owait(out_v, osem)

        for sl in range(NSL):
            c1 = accs[sl]
            u1 = c1.astype(jnp.int32)
            u0 = (sumw - c1).astype(jnp.int32)
            out_v[pl.ds(sl * LANES, LANES)] = (
                jnp.right_shift(u0 - u1, 31) & 1)
        pltpu.async_copy(
            out_v,
            out_hbm.at[pl.ds(tbase + g * BLK_COLS, CHUNK)], osem)

    @pl.when(sid == 0)
    def _():
        _start(0, sh0)

    def _outer(g2, carry):
        for b, (sh, nsh, out_v, osem) in enumerate(
                ((sh0, sh1, out_v0, osem0), (sh1, sh0, out_v1, osem1))):
            g = 2 * g2 + b

            @pl.when(sid == 0)
            def _():
                _hwait(sh)

            plsc.subcore_barrier()

            @pl.when((sid == 0) & (g + 1 < NCH))
            def _():
                _start(g + 1, nsh)

            _compute(g, sh, out_v, osem)
        return carry

    lax.fori_loop(0, NCH // 2, _outer, 0)
    _owait(out_v0, osem0)
    _owait(out_v1, osem1)


def _vote_tc_body(w_ref, x_ref, o_ref):
    x = x_ref[...].astype(jnp.float32)
    w = w_ref[...]
    c1 = jnp.sum(x * w, axis=0)
    sumw = jnp.sum(w)
    u1 = c1.astype(jnp.int32)
    u0 = (sumw - c1).astype(jnp.int32)
    o_ref[...] = (u1 > u0).astype(jnp.int32)


def _vote_tc(votes, w2):
    grid = (TC_COLS // TC_BLK,)
    off = SC_COLS // TC_BLK
    return pl.pallas_call(
        _vote_tc_body,
        grid=grid,
        in_specs=[
            pl.BlockSpec((N_VOTERS, 1), lambda j: (0, 0)),
            pl.BlockSpec((N_VOTERS, TC_BLK), lambda j: (0, j + off)),
        ],
        out_specs=pl.BlockSpec((TC_BLK,), lambda j: (j,)),
        out_shape=jax.ShapeDtypeStruct((TC_COLS,), jnp.int32),
    )(w2, votes)


def kernel(inputs, vote_weights):
    w = vote_weights.astype(jnp.float32)
    parts = []
    if SC_COLS > 0:
        wb = jnp.broadcast_to(w[:, None], (N_VOTERS, LANES))
        parts.append(_vote_sc(inputs, wb))
    if TC_COLS > 0:
        parts.append(_vote_tc(inputs, w[:, None]))
    if len(parts) == 1:
        return parts[0]
    return jnp.concatenate(parts)


# speedup vs baseline: 1.9874x; 1.9874x over previous
import functools

import jax
import jax.numpy as jnp
from jax import lax
from jax.experimental import pallas as pl
from jax.experimental.pallas import tpu as pltpu
from jax.experimental.pallas import tpu_sc as plsc

N_VOTERS = 64
BATCH = 1048576
LANES = 16
NUM_WORKERS = 32

SC_COLS = 458752
TC_COLS = BATCH - SC_COLS
CHUNK = 896
NSL = CHUNK // LANES // 2
BPW = max(SC_COLS // NUM_WORKERS, CHUNK)
NCH = BPW // CHUNK
TC_BLK = 65536

_MESH = plsc.VectorSubcoreMesh(core_axis_name="c", subcore_axis_name="s")


@functools.partial(
    pl.kernel,
    out_type=jax.ShapeDtypeStruct((max(SC_COLS, 1),), jnp.int32),
    mesh=_MESH,
    scratch_types=[
        pltpu.VMEM((N_VOTERS, LANES), jnp.float32),
        pltpu.VMEM((N_VOTERS, CHUNK), jnp.int32),
        pltpu.VMEM((N_VOTERS, CHUNK), jnp.int32),
        pltpu.VMEM((CHUNK,), jnp.int32),
        pltpu.VMEM((CHUNK,), jnp.int32),
        pltpu.SemaphoreType.DMA,
        pltpu.SemaphoreType.DMA,
        pltpu.SemaphoreType.DMA,
        pltpu.SemaphoreType.DMA,
    ],
)
def _vote_sc(votes_hbm, wb_hbm, out_hbm, wb_v, buf0, buf1, out_v0, out_v1,
             sem0, sem1, osem0, osem1):
    wid = lax.axis_index("s") * 2 + lax.axis_index("c")
    base = wid * BPW

    pltpu.sync_copy(wb_hbm, wb_v)

    def _wsum(v, s):
        return s + wb_v[v]
    sumw = lax.fori_loop(0, N_VOTERS, _wsum,
                         jnp.zeros((LANES,), jnp.float32))

    def _start(ch, buf, sem):
        pltpu.async_copy(
            votes_hbm.at[:, pl.ds(base + ch * CHUNK, CHUNK)], buf, sem)

    def _wait(buf, sem):
        pltpu.make_async_copy(
            votes_hbm.at[:, pl.ds(base, CHUNK)], buf, sem).wait()

    def _owait(ch, out_v, osem):
        pltpu.make_async_copy(
            out_v, out_hbm.at[pl.ds(base + ch * CHUNK, CHUNK)], osem).wait()

    def _compute(ch, buf, out_v, osem):
        @pl.when(ch >= 2)
        def _():
            _owait(ch, out_v, osem)

        for half in range(2):
            off = half * NSL
            def _vstep(v, accs, off=off):
                w = wb_v[v]
                return tuple(
                    accs[sl] + buf[v, pl.ds((off + sl) * LANES, LANES)]
                    .astype(jnp.float32) * w
                    for sl in range(NSL))
            zero = jnp.zeros((LANES,), jnp.float32)
            accs = lax.fori_loop(0, N_VOTERS, _vstep, (zero,) * NSL)

            for sl in range(NSL):
                c1 = accs[sl]
                u1 = c1.astype(jnp.int32)
                u0 = (sumw - c1).astype(jnp.int32)
                out_v[pl.ds((off + sl) * LANES, LANES)] = (
                    jnp.right_shift(u0 - u1, 31) & 1)
        pltpu.async_copy(
            out_v, out_hbm.at[pl.ds(base + ch * CHUNK, CHUNK)], osem)

    _start(0, buf0, sem0)

    def _outer(g2, carry):
        for b, (buf, sem, nbuf, nsem, out_v, osem) in enumerate(
                ((buf0, sem0, buf1, sem1, out_v0, osem0),
                 (buf1, sem1, buf0, sem0, out_v1, osem1))):
            ch = 2 * g2 + b

            @pl.when(ch + 1 < NCH)
            def _():
                _start(ch + 1, nbuf, nsem)

            _wait(buf, sem)
            _compute(ch, buf, out_v, osem)
        return carry

    lax.fori_loop(0, NCH // 2, _outer, 0)
    _owait(NCH - 2, out_v0, osem0)
    _owait(NCH - 1, out_v1, osem1)


def _vote_tc_body(w_ref, x_ref, o_ref):
    x = x_ref[...].astype(jnp.float32)
    w = w_ref[...]
    c1 = jnp.sum(x * w, axis=0)
    sumw = jnp.sum(w)
    u1 = c1.astype(jnp.int32)
    u0 = (sumw - c1).astype(jnp.int32)
    o_ref[...] = (u1 > u0).astype(jnp.int32)


def _vote_tc(votes, w2):
    grid = (TC_COLS // TC_BLK,)
    off = SC_COLS // TC_BLK
    return pl.pallas_call(
        _vote_tc_body,
        grid=grid,
        in_specs=[
            pl.BlockSpec((N_VOTERS, 1), lambda j: (0, 0)),
            pl.BlockSpec((N_VOTERS, TC_BLK), lambda j: (0, j + off)),
        ],
        out_specs=pl.BlockSpec((TC_BLK,), lambda j: (j,)),
        out_shape=jax.ShapeDtypeStruct((TC_COLS,), jnp.int32),
    )(w2, votes)


def kernel(inputs, vote_weights):
    w = vote_weights.astype(jnp.float32)
    parts = []
    if SC_COLS > 0:
        wb = jnp.broadcast_to(w[:, None], (N_VOTERS, LANES))
        parts.append(_vote_sc(inputs, wb))
    if TC_COLS > 0:
        parts.append(_vote_tc(inputs, w[:, None]))
    if len(parts) == 1:
        return parts[0]
    return jnp.concatenate(parts)


# speedup vs baseline: 2.0104x; 1.0116x over previous
import functools

import jax
import jax.numpy as jnp
from jax import lax
from jax.experimental import pallas as pl
from jax.experimental.pallas import tpu as pltpu
from jax.experimental.pallas import tpu_sc as plsc

N_VOTERS = 64
BATCH = 1048576
LANES = 16
NUM_WORKERS = 32

SC_COLS = 622592
TC_COLS = BATCH - SC_COLS
CHUNK = 512
NSL = CHUNK // LANES
BPW = max(SC_COLS // NUM_WORKERS, CHUNK)
NCH = BPW // CHUNK
TC_BLK = 8192

_MESH = plsc.VectorSubcoreMesh(core_axis_name="c", subcore_axis_name="s")


@functools.partial(
    pl.kernel,
    out_type=jax.ShapeDtypeStruct((max(SC_COLS, 1),), jnp.int32),
    mesh=_MESH,
    scratch_types=[
        pltpu.VMEM((N_VOTERS, LANES), jnp.float32),
        pltpu.VMEM((N_VOTERS, CHUNK), jnp.int32),
        pltpu.VMEM((N_VOTERS, CHUNK), jnp.int32),
        pltpu.VMEM((CHUNK,), jnp.int32),
        pltpu.VMEM((CHUNK,), jnp.int32),
        pltpu.SemaphoreType.DMA,
        pltpu.SemaphoreType.DMA,
        pltpu.SemaphoreType.DMA,
        pltpu.SemaphoreType.DMA,
    ],
)
def _vote_sc(votes_hbm, wb_hbm, out_hbm, wb_v, buf0, buf1, out_v0, out_v1,
             sem0, sem1, osem0, osem1):
    wid = lax.axis_index("s") * 2 + lax.axis_index("c")
    base = wid * BPW

    pltpu.sync_copy(wb_hbm, wb_v)

    def _wsum(v, s):
        return s + wb_v[v]
    sumw = lax.fori_loop(0, N_VOTERS, _wsum,
                         jnp.zeros((LANES,), jnp.float32))

    def _start(ch, buf, sem):
        pltpu.async_copy(
            votes_hbm.at[:, pl.ds(base + ch * CHUNK, CHUNK)], buf, sem)

    def _wait(buf, sem):
        pltpu.make_async_copy(
            votes_hbm.at[:, pl.ds(base, CHUNK)], buf, sem).wait()

    def _owait(ch, out_v, osem):
        pltpu.make_async_copy(
            out_v, out_hbm.at[pl.ds(base + ch * CHUNK, CHUNK)], osem).wait()

    def _compute(ch, buf, out_v, osem):
        def _vstep(v, accs):
            w = wb_v[v]
            return tuple(
                accs[sl] + buf[v, pl.ds(sl * LANES, LANES)]
                .astype(jnp.float32) * w
                for sl in range(NSL))
        zero = jnp.zeros((LANES,), jnp.float32)
        accs = lax.fori_loop(0, N_VOTERS, _vstep, (zero,) * NSL)

        @pl.when(ch >= 2)
        def _():
            _owait(ch, out_v, osem)

        for sl in range(NSL):
            c1 = accs[sl]
            u1 = c1.astype(jnp.int32)
            u0 = (sumw - c1).astype(jnp.int32)
            out_v[pl.ds(sl * LANES, LANES)] = (
                jnp.right_shift(u0 - u1, 31) & 1)
        pltpu.async_copy(
            out_v, out_hbm.at[pl.ds(base + ch * CHUNK, CHUNK)], osem)

    _start(0, buf0, sem0)

    def _outer(g2, carry):
        for b, (buf, sem, nbuf, nsem, out_v, osem) in enumerate(
                ((buf0, sem0, buf1, sem1, out_v0, osem0),
                 (buf1, sem1, buf0, sem0, out_v1, osem1))):
            ch = 2 * g2 + b

            @pl.when(ch + 1 < NCH)
            def _():
                _start(ch + 1, nbuf, nsem)

            _wait(buf, sem)
            _compute(ch, buf, out_v, osem)
        return carry

    lax.fori_loop(0, NCH // 2, _outer, 0)
    _owait(NCH - 2, out_v0, osem0)
    _owait(NCH - 1, out_v1, osem1)


def _vote_tc_body(w_ref, x_ref, o_ref):
    x = x_ref[...].astype(jnp.float32)
    w = w_ref[...]
    c1 = jnp.sum(x * w, axis=0)
    sumw = jnp.sum(w)
    u1 = c1.astype(jnp.int32)
    u0 = (sumw - c1).astype(jnp.int32)
    o_ref[...] = (u1 > u0).astype(jnp.int32)


def _vote_tc(votes, w2):
    grid = (TC_COLS // TC_BLK,)
    off = SC_COLS // TC_BLK
    return pl.pallas_call(
        _vote_tc_body,
        grid=grid,
        in_specs=[
            pl.BlockSpec((N_VOTERS, 1), lambda j: (0, 0)),
            pl.BlockSpec((N_VOTERS, TC_BLK), lambda j: (0, j + off)),
        ],
        out_specs=pl.BlockSpec((TC_BLK,), lambda j: (j,)),
        out_shape=jax.ShapeDtypeStruct((TC_COLS,), jnp.int32),
    )(w2, votes)


def kernel(inputs, vote_weights):
    w = vote_weights.astype(jnp.float32)
    parts = []
    if SC_COLS > 0:
        wb = jnp.broadcast_to(w[:, None], (N_VOTERS, LANES))
        parts.append(_vote_sc(inputs, wb))
    if TC_COLS > 0:
        parts.append(_vote_tc(inputs, w[:, None]))
    if len(parts) == 1:
        return parts[0]
    return jnp.concatenate(parts)
